# baseline (speedup 1.0000x reference)
import math

import jax
import jax.numpy as jnp
from jax import lax
from jax.experimental import pallas as pl
from jax.experimental.pallas import tpu as pltpu

B = 16384
C = 1000
N_BINS = 15
VIRTUAL_PROB = 0.9

BBLK = 4096
NBLK = B // BBLK

_Q = (1.0 - VIRTUAL_PROB) / (C - 1)
_KL_CONST = VIRTUAL_PROB * math.log(VIRTUAL_PROB) + (C - 1) * _Q * math.log(_Q)
_PMQ = VIRTUAL_PROB - _Q


def _tc_body(x_ref, tgt_ref, bnd_ref, acc1_ref, ece_ref, ent_ref, kl_ref,
             hist_ref):
    pid = pl.program_id(0)
    x = x_ref[...]
    tgt = tgt_ref[...]
    m = jnp.max(x, axis=0)
    e = jnp.exp(x - m[None, :])
    s = jnp.sum(e, axis=0)
    dot = jnp.sum(e * x, axis=0)
    sum_pred = jnp.sum(x, axis=0)
    rows = lax.broadcasted_iota(jnp.int32, (C, BBLK), 0)
    t_logit = jnp.sum(jnp.where(rows == tgt[None, :], x, 0.0), axis=0)
    lse = m + jnp.log(s)
    conf = 1.0 / s
    acc = (t_logit == m).astype(jnp.float32)
    ent_row = lse - dot / s
    kl_row = (jnp.float32(_KL_CONST)
              - jnp.float32(_Q) * (sum_pred - jnp.float32(C) * lse)
              - jnp.float32(_PMQ) * (t_logit - lse))

    cmp = (conf[:, None] > bnd_ref[...]).astype(jnp.int32)
    idx = jnp.sum(cmp[:, 1:N_BINS], axis=1)

    lanes = lax.broadcasted_iota(jnp.int32, (1, 128), 1)
    onehot = idx[:, None] == lanes
    diff = conf - acc
    contrib = jnp.sum(jnp.where(onehot, diff[:, None], 0.0), axis=0)
    scal = jnp.where(lanes[0] == 15, jnp.sum(acc),
                     jnp.where(lanes[0] == 16, jnp.sum(ent_row),
                               jnp.where(lanes[0] == 17, jnp.sum(kl_row),
                                         0.0)))
    upd = (contrib + scal)[None, :]

    @pl.when(pid == 0)
    def _():
        hist_ref[...] = upd

    @pl.when(pid > 0)
    def _():
        hist_ref[...] = hist_ref[...] + upd

    @pl.when(pid == NBLK - 1)
    def _():
        inv_b = 1.0 / B
        h = hist_ref[0, :]
        lane = lanes[0]
        ece = jnp.sum(jnp.where(lane < N_BINS, jnp.abs(h), 0.0)) * inv_b
        acc1 = jnp.sum(jnp.where(lane == 15, h, 0.0)) * inv_b
        ent = jnp.sum(jnp.where(lane == 16, h, 0.0)) * inv_b
        kl = jnp.sum(jnp.where(lane == 17, h, 0.0)) * inv_b
        acc1_ref[...] = acc1.reshape(1, 1)
        ece_ref[...] = ece.reshape(1, 1)
        ent_ref[...] = ent.reshape(1, 1)
        kl_ref[...] = kl.reshape(1, 1)


def _tc_metrics(pred_t, target, bounds):
    one_spec = pl.BlockSpec((1, 1), lambda i: (0, 0))
    return pl.pallas_call(
        _tc_body,
        grid=(NBLK,),
        in_specs=[
            pl.BlockSpec((C, BBLK), lambda i: (0, i)),
            pl.BlockSpec((BBLK,), lambda i: (i,)),
            pl.BlockSpec((1, N_BINS + 1), lambda i: (0, 0)),
        ],
        out_specs=[one_spec, one_spec, one_spec, one_spec],
        out_shape=[jax.ShapeDtypeStruct((1, 1), jnp.float32)] * 4,
        scratch_shapes=[pltpu.VMEM((1, 128), jnp.float32)],
    )(pred_t, target, bounds)


def kernel(pred, target):
    boundaries = jnp.linspace(0.0, 1.0, N_BINS + 1).astype(jnp.float32)
    bounds = boundaries.reshape(1, N_BINS + 1)
    acc1, ece, ent, kl = _tc_metrics(pred.T, target, bounds)
    return (acc1.reshape(1), ece.reshape(1), ent[0, 0], kl[0, 0])


# speedup vs baseline: 1.2776x; 1.2776x over previous
import math

import jax
import jax.numpy as jnp
from jax import lax
from jax.experimental import pallas as pl
from jax.experimental.pallas import tpu as pltpu

B = 16384
C = 1000
N_BINS = 15
VIRTUAL_PROB = 0.9

BBLK = 2048
NBLK = B // BBLK

_Q = (1.0 - VIRTUAL_PROB) / (C - 1)
_KL_CONST = VIRTUAL_PROB * math.log(VIRTUAL_PROB) + (C - 1) * _Q * math.log(_Q)
_PMQ = VIRTUAL_PROB - _Q


def _tc_body(x_ref, tgt_ref, bnd_ref, acc1_ref, ece_ref, ent_ref, kl_ref,
             hist_ref):
    pid = pl.program_id(0)
    x = x_ref[...]
    tgt = tgt_ref[...]
    m = jnp.max(x, axis=0)
    e = jnp.exp(x - m[None, :])
    ones = jnp.ones((1, C), jnp.float32)
    s = jnp.dot(ones, e, preferred_element_type=jnp.float32)[0]
    dot = jnp.dot(ones, e * x, preferred_element_type=jnp.float32)[0]
    sum_pred = jnp.dot(ones, x, preferred_element_type=jnp.float32)[0]
    rows = lax.broadcasted_iota(jnp.int32, (C, BBLK), 0)
    t_logit = jnp.sum(jnp.where(rows == tgt[None, :], x, 0.0), axis=0)
    lse = m + jnp.log(s)
    conf = 1.0 / s
    acc = (t_logit == m).astype(jnp.float32)
    ent_row = lse - dot / s
    kl_row = (jnp.float32(_KL_CONST)
              - jnp.float32(_Q) * (sum_pred - jnp.float32(C) * lse)
              - jnp.float32(_PMQ) * (t_logit - lse))

    cmp = (conf[:, None] > bnd_ref[...]).astype(jnp.int32)
    idx = jnp.sum(cmp[:, 1:N_BINS], axis=1)

    lanes = lax.broadcasted_iota(jnp.int32, (1, 128), 1)
    onehot = idx[:, None] == lanes
    diff = conf - acc
    contrib = jnp.sum(jnp.where(onehot, diff[:, None], 0.0), axis=0)
    scal = jnp.where(lanes[0] == 15, jnp.sum(acc),
                     jnp.where(lanes[0] == 16, jnp.sum(ent_row),
                               jnp.where(lanes[0] == 17, jnp.sum(kl_row),
                                         0.0)))
    upd = (contrib + scal)[None, :]

    @pl.when(pid == 0)
    def _():
        hist_ref[...] = upd

    @pl.when(pid > 0)
    def _():
        hist_ref[...] = hist_ref[...] + upd

    @pl.when(pid == NBLK - 1)
    def _():
        inv_b = 1.0 / B
        h = hist_ref[0, :]
        lane = lanes[0]
        ece = jnp.sum(jnp.where(lane < N_BINS, jnp.abs(h), 0.0)) * inv_b
        acc1 = jnp.sum(jnp.where(lane == 15, h, 0.0)) * inv_b
        ent = jnp.sum(jnp.where(lane == 16, h, 0.0)) * inv_b
        kl = jnp.sum(jnp.where(lane == 17, h, 0.0)) * inv_b
        acc1_ref[...] = acc1.reshape(1, 1)
        ece_ref[...] = ece.reshape(1, 1)
        ent_ref[...] = ent.reshape(1, 1)
        kl_ref[...] = kl.reshape(1, 1)


def _tc_metrics(pred_t, target, bounds):
    one_spec = pl.BlockSpec((1, 1), lambda i: (0, 0))
    return pl.pallas_call(
        _tc_body,
        grid=(NBLK,),
        in_specs=[
            pl.BlockSpec((C, BBLK), lambda i: (0, i)),
            pl.BlockSpec((BBLK,), lambda i: (i,)),
            pl.BlockSpec((1, N_BINS + 1), lambda i: (0, 0)),
        ],
        out_specs=[one_spec, one_spec, one_spec, one_spec],
        out_shape=[jax.ShapeDtypeStruct((1, 1), jnp.float32)] * 4,
        scratch_shapes=[pltpu.VMEM((1, 128), jnp.float32)],
    )(pred_t, target, bounds)


def kernel(pred, target):
    boundaries = jnp.linspace(0.0, 1.0, N_BINS + 1).astype(jnp.float32)
    bounds = boundaries.reshape(1, N_BINS + 1)
    acc1, ece, ent, kl = _tc_metrics(pred.T, target, bounds)
    return (acc1.reshape(1), ece.reshape(1), ent[0, 0], kl[0, 0])


# speedup vs baseline: 1.3547x; 1.0604x over previous
import math

import jax
import jax.numpy as jnp
from jax import lax
from jax.experimental import pallas as pl
from jax.experimental.pallas import tpu as pltpu

B = 16384
C = 1000
N_BINS = 15
VIRTUAL_PROB = 0.9

BBLK = 2048
NBLK = B // BBLK

_Q = (1.0 - VIRTUAL_PROB) / (C - 1)
_KL_CONST = VIRTUAL_PROB * math.log(VIRTUAL_PROB) + (C - 1) * _Q * math.log(_Q)
_PMQ = VIRTUAL_PROB - _Q


def _tc_body(x_ref, tgt_ref, bnd_ref, acc1_ref, ece_ref, ent_ref, kl_ref,
             hist_ref):
    pid = pl.program_id(0)
    x = x_ref[...]
    tgt = tgt_ref[...]
    m = jnp.max(x, axis=0)
    xm = x - m[None, :]
    e = jnp.exp(xm)
    ones = jnp.ones((1, C), jnp.float32)
    s = jnp.dot(ones, e, preferred_element_type=jnp.float32)[0]
    dot = jnp.dot(ones, e * xm, preferred_element_type=jnp.float32)[0]
    sum_pred = jnp.dot(ones, x, preferred_element_type=jnp.float32)[0]
    rows = lax.broadcasted_iota(jnp.int32, (C, BBLK), 0)
    t_xm = jnp.sum(jnp.where(rows == tgt[None, :], xm, 0.0), axis=0)
    logs = jnp.log(s)
    lse = m + logs
    conf = 1.0 / s
    acc = (t_xm == 0.0).astype(jnp.float32)
    ent_row = logs - dot / s
    kl_row = (jnp.float32(_KL_CONST)
              - jnp.float32(_Q) * (sum_pred - jnp.float32(C) * lse)
              - jnp.float32(_PMQ) * (t_xm - logs))

    cmp = (conf[:, None] > bnd_ref[...]).astype(jnp.int32)
    idx = jnp.sum(cmp[:, 1:N_BINS], axis=1)

    lanes = lax.broadcasted_iota(jnp.int32, (1, 128), 1)
    onehot = idx[:, None] == lanes
    diff = conf - acc
    contrib = jnp.sum(jnp.where(onehot, diff[:, None], 0.0), axis=0)
    scal = jnp.where(lanes[0] == 15, jnp.sum(acc),
                     jnp.where(lanes[0] == 16, jnp.sum(ent_row),
                               jnp.where(lanes[0] == 17, jnp.sum(kl_row),
                                         0.0)))
    upd = (contrib + scal)[None, :]

    @pl.when(pid == 0)
    def _():
        hist_ref[...] = upd

    @pl.when(pid > 0)
    def _():
        hist_ref[...] = hist_ref[...] + upd

    @pl.when(pid == NBLK - 1)
    def _():
        inv_b = 1.0 / B
        h = hist_ref[0, :]
        lane = lanes[0]
        ece = jnp.sum(jnp.where(lane < N_BINS, jnp.abs(h), 0.0)) * inv_b
        acc1 = jnp.sum(jnp.where(lane == 15, h, 0.0)) * inv_b
        ent = jnp.sum(jnp.where(lane == 16, h, 0.0)) * inv_b
        kl = jnp.sum(jnp.where(lane == 17, h, 0.0)) * inv_b
        acc1_ref[...] = acc1.reshape(1, 1)
        ece_ref[...] = ece.reshape(1, 1)
        ent_ref[...] = ent.reshape(1, 1)
        kl_ref[...] = kl.reshape(1, 1)


def _tc_metrics(pred_t, target, bounds):
    one_spec = pl.BlockSpec((1, 1), lambda i: (0, 0))
    return pl.pallas_call(
        _tc_body,
        grid=(NBLK,),
        in_specs=[
            pl.BlockSpec((C, BBLK), lambda i: (0, i)),
            pl.BlockSpec((BBLK,), lambda i: (i,)),
            pl.BlockSpec((1, N_BINS + 1), lambda i: (0, 0)),
        ],
        out_specs=[one_spec, one_spec, one_spec, one_spec],
        out_shape=[jax.ShapeDtypeStruct((1, 1), jnp.float32)] * 4,
        scratch_shapes=[pltpu.VMEM((1, 128), jnp.float32)],
    )(pred_t, target, bounds)


def kernel(pred, target):
    boundaries = jnp.linspace(0.0, 1.0, N_BINS + 1).astype(jnp.float32)
    bounds = boundaries.reshape(1, N_BINS + 1)
    acc1, ece, ent, kl = _tc_metrics(pred.T, target, bounds)
    return (acc1.reshape(1), ece.reshape(1), ent[0, 0], kl[0, 0])
